# Initial kernel scaffold; baseline (speedup 1.0000x reference)
import jax
import jax.numpy as jnp
from jax.experimental import pallas as pl


def kernel(encoder_out, align_phone, text_phone):
    raise NotImplementedError("write your pallas kernel here")


# speedup vs baseline: 51.3400x; 51.3400x over previous
import functools

import jax
import jax.numpy as jnp
from jax import lax
from jax.experimental import pallas as pl
from jax.experimental.pallas import tpu as pltpu
from jax.experimental.pallas import tpu_sc as plsc

_INFO = plsc.get_sparse_core_info()
_NC, _NS, _L = _INFO.num_cores, _INFO.num_subcores, _INFO.num_lanes

_CHUNK = 128


def _make_kernel(B, TT, TA, D):
    mesh = plsc.VectorSubcoreMesh(core_axis_name="c", subcore_axis_name="s")
    half = TA // 2
    n_chunks = half // _CHUNK
    b_per_core = B // _NC

    @functools.partial(
        pl.kernel,
        out_type=jax.ShapeDtypeStruct((B * TA, D), jnp.float32),
        mesh=mesh,
        scratch_types=[
            pltpu.VMEM((B, TA), jnp.int32),
            pltpu.VMEM((B, TT), jnp.int32),
            pltpu.VMEM((B, TA), jnp.int32),
            pltpu.VMEM((B, _L), jnp.int32),
            pltpu.VMEM((_CHUNK,), jnp.int32),
            pltpu.VMEM((_CHUNK, D), jnp.float32),
            pltpu.VMEM_SHARED((B, TA), jnp.int32),
            pltpu.VMEM_SHARED((B, _L), jnp.int32),
            pltpu.VMEM_SHARED((_CHUNK, D), jnp.float32),
            pltpu.SemaphoreType.DMA,
        ],
        compiler_params=pltpu.CompilerParams(
            use_tc_tiling_on_sc=False, needs_layout_passes=False),
    )
    def aligner(enc_hbm, align_hbm, text_hbm, out_hbm,
                align_v, text_v, idxb_v, live_v, idxc_v, rows_v,
                idx_sh, live_sh, zeros_sh, sem):
        c = lax.axis_index("c")
        s = lax.axis_index("s")
        lane = lax.iota(jnp.int32, _L)
        zeros_i = jnp.zeros((_L,), jnp.int32)
        ones_i = jnp.ones((_L,), jnp.int32)
        tt_max = jnp.full((_L,), TT - 1, jnp.int32)
        tt_full = jnp.full((_L,), TT, jnp.int32)
        zf = jnp.zeros((_L,), jnp.float32)

        @pl.when(s == 0)
        def _phase1():
            pltpu.sync_copy(align_hbm, align_v)
            pltpu.sync_copy(text_hbm, text_v)
            lane_base = lane * TT
            before0 = plsc.load_gather(text_v, [lane, zeros_i])
            plsc.store_scatter(idxb_v, [lane, zeros_i], lane_base)

            def step(t, carry):
                ind, before, live = carry
                tvec = jnp.full((_L,), t, jnp.int32)
                a = plsc.load_gather(align_v, [lane, tvec])
                ind = ind + jnp.where(a == before, zeros_i, ones_i)
                safe = jnp.minimum(ind, tt_max)
                before = plsc.load_gather(text_v, [lane, safe])
                plsc.store_scatter(idxb_v, [lane, tvec], lane_base + safe)
                live = live + jnp.where(ind < tt_full, ones_i, zeros_i)
                return ind, before, live

            _, _, live = lax.fori_loop(1, TA, step, (zeros_i, before0, ones_i))
            for j in range(_L):
                plsc.store_scatter(
                    live_v, [lane, jnp.full((_L,), j, jnp.int32)], live)
            pltpu.sync_copy(idxb_v, idx_sh)
            pltpu.sync_copy(live_v, live_sh)

        @pl.when(s == 1)
        def _make_zeros():
            def zrow(r, carry):
                for j in range(D // _L):
                    rows_v[r, pl.ds(j * _L, _L)] = zf
                return carry

            lax.fori_loop(0, _CHUNK, zrow, 0)
            pltpu.sync_copy(rows_v, zeros_sh)

        plsc.subcore_barrier()

        b = c * b_per_core + s // 2
        f_base = (s % 2) * half
        pltpu.sync_copy(live_sh, live_v)
        live_b = live_v[b][0]

        def do_chunk(k, carry):
            f0 = f_base + k * _CHUNK
            gbase = b * TA + f0
            rem = live_b - f0

            @pl.when(rem > 0)
            def _gather():
                pltpu.sync_copy(idx_sh.at[b, pl.ds(f0, _CHUNK)], idxc_v)
                pltpu.async_copy(enc_hbm.at[idxc_v], rows_v, sem).wait()

                @pl.when(rem < _CHUNK)
                def _zero_tail():
                    def zrow(r, carry2):
                        for j in range(D // _L):
                            rows_v[r, pl.ds(j * _L, _L)] = zf
                        return carry2

                    lax.fori_loop(rem, _CHUNK, zrow, 0)

                pltpu.sync_copy(rows_v, out_hbm.at[pl.ds(gbase, _CHUNK)])

            @pl.when(rem <= 0)
            def _zeros():
                pltpu.sync_copy(zeros_sh, out_hbm.at[pl.ds(gbase, _CHUNK)])

            return carry

        lax.fori_loop(0, n_chunks, do_chunk, 0)

    return aligner


def kernel(encoder_out, align_phone, text_phone):
    B, TT, D = encoder_out.shape
    TA = align_phone.shape[1]
    enc_flat = encoder_out.reshape(B * TT, D)
    out = _make_kernel(B, TT, TA, D)(
        enc_flat, align_phone.astype(jnp.int32), text_phone.astype(jnp.int32))
    return out.reshape(B, TA, D)


# speedup vs baseline: 55.6070x; 1.0831x over previous
import functools

import jax
import jax.numpy as jnp
from jax import lax
from jax.experimental import pallas as pl
from jax.experimental.pallas import tpu as pltpu
from jax.experimental.pallas import tpu_sc as plsc

_INFO = plsc.get_sparse_core_info()
_NC, _NS, _L = _INFO.num_cores, _INFO.num_subcores, _INFO.num_lanes

_CHUNK = 128


def _make_kernel(B, TT, TA, D):
    mesh = plsc.VectorSubcoreMesh(core_axis_name="c", subcore_axis_name="s")
    half = TA // 2
    n_chunks = half // _CHUNK
    b_per_core = B // _NC

    @functools.partial(
        pl.kernel,
        out_type=jax.ShapeDtypeStruct((B * TA, D), jnp.float32),
        mesh=mesh,
        scratch_types=[
            pltpu.VMEM((B, TA), jnp.int32),
            pltpu.VMEM((B, TT), jnp.int32),
            pltpu.VMEM((B, TA), jnp.int32),
            pltpu.VMEM((B, _L), jnp.int32),
            pltpu.VMEM((_CHUNK,), jnp.int32),
            pltpu.VMEM((_CHUNK, D), jnp.float32),
            pltpu.VMEM_SHARED((B, TA), jnp.int32),
            pltpu.VMEM_SHARED((B, _L), jnp.int32),
            pltpu.VMEM_SHARED((_CHUNK, D), jnp.float32),
            pltpu.SemaphoreType.DMA,
        ],
        compiler_params=pltpu.CompilerParams(
            use_tc_tiling_on_sc=False, needs_layout_passes=False),
    )
    def aligner(enc_hbm, align_hbm, text_hbm, out_hbm,
                align_v, text_v, idxb_v, live_v, idxc_v, rows_v,
                idx_sh, live_sh, zeros_sh, sem):
        c = lax.axis_index("c")
        s = lax.axis_index("s")
        lane = lax.iota(jnp.int32, _L)
        zeros_i = jnp.zeros((_L,), jnp.int32)
        ones_i = jnp.ones((_L,), jnp.int32)
        tt_max = jnp.full((_L,), TT - 1, jnp.int32)
        tt_full = jnp.full((_L,), TT, jnp.int32)
        zf = jnp.zeros((_L,), jnp.float32)

        @pl.when(s == 0)
        def _phase1():
            pltpu.sync_copy(align_hbm, align_v)
            pltpu.sync_copy(text_hbm, text_v)
            plsc.store_scatter(idxb_v, [lane, zeros_i], zeros_i)

            def step(t, carry):
                ind, t0, t1, t2, live = carry
                tvec = jnp.full((_L,), t, jnp.int32)
                a = plsc.load_gather(align_v, [lane, tvec])
                m = a == t0
                ind = jnp.where(m, ind, ind + ones_i)
                t0 = jnp.where(m, t0, t1)
                t1 = jnp.where(m, t1, t2)
                safe = jnp.minimum(ind, tt_max)
                t2 = plsc.load_gather(
                    text_v, [lane, jnp.minimum(safe + 2, tt_max)])
                plsc.store_scatter(idxb_v, [lane, tvec], safe)
                live = live + jnp.where(ind < tt_full, ones_i, zeros_i)
                return ind, t0, t1, t2, live

            init = (zeros_i,
                    plsc.load_gather(text_v, [lane, zeros_i]),
                    plsc.load_gather(
                        text_v, [lane, jnp.minimum(ones_i, tt_max)]),
                    plsc.load_gather(
                        text_v, [lane, jnp.minimum(ones_i + ones_i, tt_max)]),
                    ones_i)
            n_main = ((TA - 1) // 8) * 8
            carry = plsc.parallel_loop(1, 1 + n_main, carry=init, unroll=8)(
                step)
            carry = plsc.parallel_loop(1 + n_main, TA, carry=carry)(step)
            live = carry[4]
            for j in range(_L):
                plsc.store_scatter(
                    live_v, [lane, jnp.full((_L,), j, jnp.int32)], live)
            pltpu.sync_copy(idxb_v, idx_sh)
            pltpu.sync_copy(live_v, live_sh)

        @pl.when(s == 1)
        def _make_zeros():
            def zrow(r, carry):
                for j in range(D // _L):
                    rows_v[r, pl.ds(j * _L, _L)] = zf
                return carry

            lax.fori_loop(0, _CHUNK, zrow, 0)
            pltpu.sync_copy(rows_v, zeros_sh)

        plsc.subcore_barrier()

        b = c * b_per_core + s // 2
        f_base = (s % 2) * half
        pltpu.sync_copy(live_sh, live_v)
        live_b = live_v[b][0]

        def do_chunk(k, carry):
            f0 = f_base + k * _CHUNK
            gbase = b * TA + f0
            rem = live_b - f0

            @pl.when(rem > 0)
            def _gather():
                pltpu.sync_copy(idx_sh.at[b, pl.ds(f0, _CHUNK)], idxc_v)
                boff = jnp.full((_L,), b * TT, jnp.int32)
                for j in range(_CHUNK // _L):
                    sl = pl.ds(j * _L, _L)
                    idxc_v[sl] = idxc_v[sl] + boff
                pltpu.async_copy(enc_hbm.at[idxc_v], rows_v, sem).wait()

                @pl.when(rem < _CHUNK)
                def _zero_tail():
                    def zrow(r, carry2):
                        for j in range(D // _L):
                            rows_v[r, pl.ds(j * _L, _L)] = zf
                        return carry2

                    lax.fori_loop(rem, _CHUNK, zrow, 0)

                pltpu.sync_copy(rows_v, out_hbm.at[pl.ds(gbase, _CHUNK)])

            @pl.when(rem <= 0)
            def _zeros():
                pltpu.sync_copy(zeros_sh, out_hbm.at[pl.ds(gbase, _CHUNK)])

            return carry

        lax.fori_loop(0, n_chunks, do_chunk, 0)

    return aligner


def kernel(encoder_out, align_phone, text_phone):
    B, TT, D = encoder_out.shape
    TA = align_phone.shape[1]
    enc_flat = encoder_out.reshape(B * TT, D)
    out = _make_kernel(B, TT, TA, D)(
        enc_flat, align_phone.astype(jnp.int32), text_phone.astype(jnp.int32))
    return out.reshape(B, TA, D)


# speedup vs baseline: 83.9958x; 1.5105x over previous
import functools

import jax
import jax.numpy as jnp
from jax import lax
from jax.experimental import pallas as pl
from jax.experimental.pallas import tpu as pltpu
from jax.experimental.pallas import tpu_sc as plsc

_INFO = plsc.get_sparse_core_info()
_NC, _NS, _L = _INFO.num_cores, _INFO.num_subcores, _INFO.num_lanes

_CHUNK = 128


def _make_kernel(B, TT, TA, D):
    mesh = plsc.VectorSubcoreMesh(core_axis_name="c", subcore_axis_name="s")
    half = TA // 2
    n_chunks = half // _CHUNK
    b_per_core = B // _NC

    KD = D // 128

    @functools.partial(
        pl.kernel,
        out_type=jax.ShapeDtypeStruct((B * TA * KD, 128), jnp.float32),
        mesh=mesh,
        scratch_types=[
            pltpu.VMEM((B, TA), jnp.int32),
            pltpu.VMEM((B, TT), jnp.int32),
            pltpu.VMEM((B, TA), jnp.int32),
            pltpu.VMEM((B, _L), jnp.int32),
            pltpu.VMEM((_CHUNK,), jnp.int32),
            pltpu.VMEM((KD, _CHUNK), jnp.int32),
            pltpu.VMEM((KD * _CHUNK, 128), jnp.float32),
            pltpu.VMEM_SHARED((B, TA), jnp.int32),
            pltpu.VMEM_SHARED((B, _L), jnp.int32),
            pltpu.VMEM_SHARED((KD * _CHUNK, 128), jnp.float32),
            pltpu.SemaphoreType.DMA,
        ],
        compiler_params=pltpu.CompilerParams(
            use_tc_tiling_on_sc=False, needs_layout_passes=False),
    )
    def aligner(enc_hbm, align_hbm, text_hbm, out_hbm,
                align_v, text_v, idxb_v, live_v, idxc_v, idx2_v, rows_v,
                idx_sh, live_sh, zeros_sh, sem):
        c = lax.axis_index("c")
        s = lax.axis_index("s")
        lane = lax.iota(jnp.int32, _L)
        zeros_i = jnp.zeros((_L,), jnp.int32)
        ones_i = jnp.ones((_L,), jnp.int32)
        tt_max = jnp.full((_L,), TT - 1, jnp.int32)
        tt_full = jnp.full((_L,), TT, jnp.int32)
        zf = jnp.zeros((_L,), jnp.float32)

        @pl.when(s == 0)
        def _phase1():
            pltpu.sync_copy(align_hbm, align_v)
            pltpu.sync_copy(text_hbm, text_v)
            plsc.store_scatter(idxb_v, [lane, zeros_i], zeros_i)

            def step(t, carry):
                ind, t0, t1, t2, live = carry
                tvec = jnp.full((_L,), t, jnp.int32)
                a = plsc.load_gather(align_v, [lane, tvec])
                m = a == t0
                ind = jnp.where(m, ind, ind + ones_i)
                t0 = jnp.where(m, t0, t1)
                t1 = jnp.where(m, t1, t2)
                safe = jnp.minimum(ind, tt_max)
                t2 = plsc.load_gather(
                    text_v, [lane, jnp.minimum(safe + 2, tt_max)])
                plsc.store_scatter(idxb_v, [lane, tvec], safe)
                live = live + jnp.where(ind < tt_full, ones_i, zeros_i)
                return ind, t0, t1, t2, live

            init = (zeros_i,
                    plsc.load_gather(text_v, [lane, zeros_i]),
                    plsc.load_gather(
                        text_v, [lane, jnp.minimum(ones_i, tt_max)]),
                    plsc.load_gather(
                        text_v, [lane, jnp.minimum(ones_i + ones_i, tt_max)]),
                    ones_i)
            n_main = ((TA - 1) // 8) * 8
            carry = plsc.parallel_loop(1, 1 + n_main, carry=init, unroll=8)(
                step)
            carry = plsc.parallel_loop(1 + n_main, TA, carry=carry)(step)
            live = carry[4]
            for j in range(_L):
                plsc.store_scatter(
                    live_v, [lane, jnp.full((_L,), j, jnp.int32)], live)
            pltpu.sync_copy(idxb_v, idx_sh)
            pltpu.sync_copy(live_v, live_sh)

        @pl.when(s == 1)
        def _make_zeros():
            def zrow(r, carry):
                for j in range(128 // _L):
                    rows_v[r, pl.ds(j * _L, _L)] = zf
                return carry

            lax.fori_loop(0, KD * _CHUNK, zrow, 0)
            pltpu.sync_copy(rows_v, zeros_sh)

        plsc.subcore_barrier()

        b = c * b_per_core + s // 2
        f_base = (s % 2) * half
        pltpu.sync_copy(live_sh, live_v)
        live_b = live_v[b][0]

        lane7 = lane & jnp.full((_L,), 7, jnp.int32)
        lane_k8 = lane & jnp.full((_L,), 8, jnp.int32)

        def do_chunk(k, carry):
            f0 = f_base + k * _CHUNK
            obase = KD * (b * TA + f0)
            rem = live_b - f0

            @pl.when(rem > 0)
            def _gather():
                pltpu.sync_copy(idx_sh.at[b, pl.ds(f0, _CHUNK)], idxc_v)
                boff = jnp.full((_L,), b * TT * KD, jnp.int32)
                for g in range(_CHUNK // 8):
                    v = plsc.load_gather(
                        idxc_v, [jnp.full((_L,), g * 8, jnp.int32) + lane7])
                    row = boff + ((v >> 3) << 4) + (v & 7) + lane_k8
                    idx2_v[g // 8, pl.ds((g % 8) * _L, _L)] = row
                cps = [
                    pltpu.async_copy(
                        enc_hbm.at[idx2_v.at[kk]],
                        rows_v.at[pl.ds(kk * _CHUNK, _CHUNK)], sem)
                    for kk in range(KD)
                ]
                for cp in cps:
                    cp.wait()

                @pl.when(rem < _CHUNK)
                def _zero_tail():
                    def zrow(p, carry2):
                        frame = (p // (8 * KD)) * 8 + (p % 8)

                        @pl.when(frame >= rem)
                        def _z():
                            for j in range(128 // _L):
                                rows_v[p, pl.ds(j * _L, _L)] = zf

                        return carry2

                    lax.fori_loop((rem // 8) * 8 * KD, KD * _CHUNK, zrow, 0)

                pltpu.sync_copy(rows_v, out_hbm.at[pl.ds(obase, KD * _CHUNK)])

            @pl.when(rem <= 0)
            def _zeros():
                pltpu.sync_copy(zeros_sh, out_hbm.at[pl.ds(obase, KD * _CHUNK)])

            return carry

        lax.fori_loop(0, n_chunks, do_chunk, 0)

    return aligner


def kernel(encoder_out, align_phone, text_phone):
    B, TT, D = encoder_out.shape
    TA = align_phone.shape[1]
    KD = D // 128
    enc_t = (encoder_out.reshape(B, TT // 8, 8, KD, 128)
             .transpose(0, 1, 3, 2, 4).reshape(B * TT * KD, 128))
    out = _make_kernel(B, TT, TA, D)(
        enc_t, align_phone.astype(jnp.int32), text_phone.astype(jnp.int32))
    return (out.reshape(B, TA // 8, KD, 8, 128)
            .transpose(0, 1, 3, 2, 4).reshape(B, TA, D))


# speedup vs baseline: 120.8608x; 1.4389x over previous
import functools

import jax
import jax.numpy as jnp
from jax import lax
from jax.experimental import pallas as pl
from jax.experimental.pallas import tpu as pltpu
from jax.experimental.pallas import tpu_sc as plsc

_INFO = plsc.get_sparse_core_info()
_NC, _NS, _L = _INFO.num_cores, _INFO.num_subcores, _INFO.num_lanes

_CHUNK = 128


def _make_kernel(B, TT, TA, D):
    mesh = plsc.VectorSubcoreMesh(core_axis_name="c", subcore_axis_name="s")
    half = TA // 2
    n_chunks = half // _CHUNK
    b_per_core = B // _NC

    KD = D // 128

    @functools.partial(
        pl.kernel,
        out_type=jax.ShapeDtypeStruct((B * TA * KD, 128), jnp.float32),
        mesh=mesh,
        scratch_types=[
            pltpu.VMEM((TA, B), jnp.int32),
            pltpu.VMEM((TT, B), jnp.int32),
            pltpu.VMEM((TA, B), jnp.int32),
            pltpu.VMEM((B, _L), jnp.int32),
            pltpu.VMEM((_CHUNK, 1), jnp.int32),
            pltpu.VMEM((KD, _CHUNK), jnp.int32),
            pltpu.VMEM((KD * _CHUNK, 128), jnp.float32),
            pltpu.VMEM_SHARED((TA, B), jnp.int32),
            pltpu.VMEM_SHARED((B, _L), jnp.int32),
            pltpu.VMEM_SHARED((KD * _CHUNK, 128), jnp.float32),
            pltpu.SemaphoreType.DMA,
        ],
        compiler_params=pltpu.CompilerParams(
            use_tc_tiling_on_sc=False, needs_layout_passes=False),
    )
    def aligner(enc_hbm, align_hbm, text_hbm, out_hbm,
                align_v, text_v, idxb_v, live_v, idxc_v, idx2_v, rows_v,
                idx_sh, live_sh, zeros_sh, sem):
        c = lax.axis_index("c")
        s = lax.axis_index("s")
        lane = lax.iota(jnp.int32, _L)
        zeros_i = jnp.zeros((_L,), jnp.int32)
        ones_i = jnp.ones((_L,), jnp.int32)
        tt_max = jnp.full((_L,), TT - 1, jnp.int32)
        tt_full = jnp.full((_L,), TT, jnp.int32)
        zf = jnp.zeros((_L,), jnp.float32)

        @pl.when(s == 0)
        def _phase1():
            pltpu.sync_copy(align_hbm, align_v)
            pltpu.sync_copy(text_hbm, text_v)
            idxb_v[0] = zeros_i

            DEPTH = 5

            def step(t, carry):
                ind, ts, live = carry
                a = align_v[t]
                m = a == ts[0]
                ind = jnp.where(m, ind, ind + ones_i)
                ts = [jnp.where(m, ts[k], ts[k + 1])
                      for k in range(DEPTH - 1)] + [None]
                safe = jnp.minimum(ind, tt_max)
                ts[DEPTH - 1] = plsc.load_gather(
                    text_v,
                    [jnp.minimum(safe + (DEPTH - 1), tt_max), lane])
                idxb_v[t] = safe
                live = live + jnp.where(ind < tt_full, ones_i, zeros_i)
                return ind, ts, live

            init = (zeros_i,
                    [plsc.load_gather(
                        text_v,
                        [jnp.minimum(zeros_i + k, tt_max), lane])
                     for k in range(DEPTH)],
                    ones_i)
            n_main = ((TA - 1) // 8) * 8
            carry = plsc.parallel_loop(1, 1 + n_main, carry=init, unroll=8)(
                step)
            carry = plsc.parallel_loop(1 + n_main, TA, carry=carry)(step)
            live = carry[2]
            for j in range(_L):
                plsc.store_scatter(
                    live_v, [lane, jnp.full((_L,), j, jnp.int32)], live)
            pltpu.sync_copy(idxb_v, idx_sh)
            pltpu.sync_copy(live_v, live_sh)

        @pl.when(s == 1)
        def _make_zeros():
            def zrow(r, carry):
                for j in range(128 // _L):
                    rows_v[r, pl.ds(j * _L, _L)] = zf
                return carry

            lax.fori_loop(0, KD * _CHUNK, zrow, 0)
            pltpu.sync_copy(rows_v, zeros_sh)

        plsc.subcore_barrier()

        b = c * b_per_core + s // 2
        f_base = (s % 2) * half
        pltpu.sync_copy(live_sh, live_v)
        live_b = live_v[b][0]

        lane7 = lane & jnp.full((_L,), 7, jnp.int32)
        lane_k8 = lane & jnp.full((_L,), 8, jnp.int32)

        def do_chunk(k, carry):
            f0 = f_base + k * _CHUNK
            obase = KD * (b * TA + f0)
            rem = live_b - f0

            @pl.when(rem > 0)
            def _gather():
                pltpu.sync_copy(
                    idx_sh.at[pl.ds(f0, _CHUNK), pl.ds(b, 1)], idxc_v)
                boff = jnp.full((_L,), b * TT * KD, jnp.int32)
                for g in range(_CHUNK // 8):
                    v = plsc.load_gather(
                        idxc_v,
                        [jnp.full((_L,), g * 8, jnp.int32) + lane7, zeros_i])
                    row = boff + ((v >> 3) << 4) + (v & 7) + lane_k8
                    idx2_v[g // 8, pl.ds((g % 8) * _L, _L)] = row
                cps = [
                    pltpu.async_copy(
                        enc_hbm.at[idx2_v.at[kk]],
                        rows_v.at[pl.ds(kk * _CHUNK, _CHUNK)], sem)
                    for kk in range(KD)
                ]
                for cp in cps:
                    cp.wait()

                @pl.when(rem < _CHUNK)
                def _zero_tail():
                    def zrow(p, carry2):
                        frame = (p // (8 * KD)) * 8 + (p % 8)

                        @pl.when(frame >= rem)
                        def _z():
                            for j in range(128 // _L):
                                rows_v[p, pl.ds(j * _L, _L)] = zf

                        return carry2

                    lax.fori_loop((rem // 8) * 8 * KD, KD * _CHUNK, zrow, 0)

                pltpu.sync_copy(rows_v, out_hbm.at[pl.ds(obase, KD * _CHUNK)])

            @pl.when(rem <= 0)
            def _zeros():
                pltpu.sync_copy(zeros_sh, out_hbm.at[pl.ds(obase, KD * _CHUNK)])

            return carry

        lax.fori_loop(0, n_chunks, do_chunk, 0)

    return aligner


def kernel(encoder_out, align_phone, text_phone):
    B, TT, D = encoder_out.shape
    TA = align_phone.shape[1]
    KD = D // 128
    enc_t = (encoder_out.reshape(B, TT // 8, 8, KD, 128)
             .transpose(0, 1, 3, 2, 4).reshape(B * TT * KD, 128))
    out = _make_kernel(B, TT, TA, D)(
        enc_t, align_phone.astype(jnp.int32).T, text_phone.astype(jnp.int32).T)
    return (out.reshape(B, TA // 8, KD, 8, 128)
            .transpose(0, 1, 3, 2, 4).reshape(B, TA, D))
